# Initial kernel scaffold; baseline (speedup 1.0000x reference)
import jax
import jax.numpy as jnp
from jax.experimental import pallas as pl


def kernel(feats, edge_index, gc_W, gc_b, res_W, res_b, bn1_g, bn1_b, gate_W, gate_b, w1, b1, w2, b2, bn2_g, bn2_b):
    raise NotImplementedError("write your pallas kernel here")


# speedup vs baseline: 1.3440x; 1.3440x over previous
import functools

import jax
import jax.numpy as jnp
from jax import lax
from jax.experimental import pallas as pl
from jax.experimental.pallas import tpu as pltpu
from jax.experimental.pallas import tpu_sc as plsc

F32 = jnp.float32
EPS = 1e-5


def _gather_sc(table, idx):
    T, Dc = table.shape
    P = idx.shape[0]
    RPT = P // 32
    CH = 128
    while RPT % CH:
        CH -= 8
    NCH = RPT // CH

    mesh = plsc.VectorSubcoreMesh(core_axis_name="c", subcore_axis_name="s")

    @functools.partial(
        pl.kernel,
        mesh=mesh,
        out_type=jax.ShapeDtypeStruct((P, Dc), F32),
        scratch_types=[
            pltpu.VMEM((RPT,), jnp.int32),
            pltpu.VMEM((CH, Dc), F32),
            pltpu.SemaphoreType.DMA,
        ],
    )
    def k(table_hbm, idx_hbm, out_hbm, idx_v, rows_v, sem):
        wid = lax.axis_index("s") * 2 + lax.axis_index("c")
        b0 = wid * RPT
        pltpu.sync_copy(idx_hbm.at[pl.ds(b0, RPT)], idx_v)

        def chunk(j, _):
            iv = idx_v.at[pl.ds(j * CH, CH)]
            pltpu.async_copy(table_hbm.at[iv], rows_v, sem).wait()
            pltpu.sync_copy(rows_v, out_hbm.at[pl.ds(b0 + j * CH, CH)])
            return 0

        lax.fori_loop(0, NCH, chunk, 0)

    return k(table, idx)


def _tc_support_res(feats, gc_W, res_W, res_b, bm):
    N, D = feats.shape

    def body(feats_ref, gcw_ref, resw_ref, rb_ref, sup_ref, res_ref):
        x = feats_ref[...]
        sup_ref[...] = jnp.dot(x, gcw_ref[...], preferred_element_type=F32)
        res_ref[...] = jnp.dot(x, resw_ref[...],
                               preferred_element_type=F32) + rb_ref[...]

    return pl.pallas_call(
        body,
        grid=(N // bm,),
        in_specs=[
            pl.BlockSpec((bm, D), lambda i: (i, 0)),
            pl.BlockSpec((D, D), lambda i: (0, 0)),
            pl.BlockSpec((D, D), lambda i: (0, 0)),
            pl.BlockSpec((1, D), lambda i: (0, 0)),
        ],
        out_specs=[
            pl.BlockSpec((bm, D), lambda i: (i, 0)),
            pl.BlockSpec((bm, D), lambda i: (i, 0)),
        ],
        out_shape=[
            jax.ShapeDtypeStruct((N, D), F32),
            jax.ShapeDtypeStruct((N, D), F32),
        ],
    )(feats, gc_W, res_W, res_b.reshape(1, D))


def _tc_zsum(agg, res, gc_b, bm):
    N, D = agg.shape

    def body(a_ref, r_ref, b_ref, z_ref, st_ref):
        z = a_ref[...] + b_ref[...] + r_ref[...]
        z_ref[...] = z
        part = jnp.concatenate([jnp.sum(z, axis=0, keepdims=True),
                                jnp.zeros((7, D), F32)], axis=0)

        @pl.when(pl.program_id(0) == 0)
        def _():
            st_ref[...] = part

        @pl.when(pl.program_id(0) > 0)
        def _():
            st_ref[...] = st_ref[...] + part

    return pl.pallas_call(
        body,
        grid=(N // bm,),
        in_specs=[
            pl.BlockSpec((bm, D), lambda i: (i, 0)),
            pl.BlockSpec((bm, D), lambda i: (i, 0)),
            pl.BlockSpec((1, D), lambda i: (0, 0)),
        ],
        out_specs=[
            pl.BlockSpec((bm, D), lambda i: (i, 0)),
            pl.BlockSpec((8, D), lambda i: (0, 0)),
        ],
        out_shape=[
            jax.ShapeDtypeStruct((N, D), F32),
            jax.ShapeDtypeStruct((8, D), F32),
        ],
    )(agg, res, gc_b.reshape(1, D))


def _tc_varsum(z, mu, bm):
    N, D = z.shape

    def body(z_ref, m_ref, st_ref):
        dzz = z_ref[...] - m_ref[...]
        part = jnp.concatenate([jnp.sum(dzz * dzz, axis=0, keepdims=True),
                                jnp.zeros((7, D), F32)], axis=0)

        @pl.when(pl.program_id(0) == 0)
        def _():
            st_ref[...] = part

        @pl.when(pl.program_id(0) > 0)
        def _():
            st_ref[...] = st_ref[...] + part

    return pl.pallas_call(
        body,
        grid=(N // bm,),
        in_specs=[
            pl.BlockSpec((bm, D), lambda i: (i, 0)),
            pl.BlockSpec((1, D), lambda i: (0, 0)),
        ],
        out_specs=pl.BlockSpec((8, D), lambda i: (0, 0)),
        out_shape=jax.ShapeDtypeStruct((8, D), F32),
    )(z, mu)


def _tc_bn_gate(z, a1, c1, gate_W, gate_b, bm):
    N, D = z.shape
    NE = gate_W.shape[1]
    grid = N // bm

    def body(z_ref, a_ref, c_ref, gw_ref, gb_ref, new_ref, ei_ref, gv_ref):
        new = z_ref[...] * a_ref[...] + c_ref[...]
        new_ref[...] = new
        lg = jnp.dot(new, gw_ref[...], preferred_element_type=F32)
        lg = lg + gb_ref[...]
        io = lax.broadcasted_iota(jnp.int32, lg.shape, 1)
        m1 = jnp.max(lg, axis=1, keepdims=True)
        i1 = jnp.min(jnp.where(lg == m1, io, NE), axis=1, keepdims=True)
        lg2 = jnp.where(io == i1, -jnp.inf, lg)
        m2 = jnp.max(lg2, axis=1, keepdims=True)
        i2 = jnp.min(jnp.where(lg2 == m2, io, NE), axis=1, keepdims=True)
        g1 = 1.0 / (1.0 + jnp.exp(m2 - m1))
        g2 = 1.0 - g1
        cio = lax.broadcasted_iota(jnp.int32, (ei_ref.shape[0], 128), 1)
        ei_ref[...] = jnp.where(cio == 0, i1, jnp.where(cio == 1, i2, 0))
        gv_ref[...] = jnp.where(cio == 0, g1, jnp.where(cio == 1, g2, 0.0))

    return pl.pallas_call(
        body,
        grid=(grid,),
        in_specs=[
            pl.BlockSpec((bm, D), lambda i: (i, 0)),
            pl.BlockSpec((1, D), lambda i: (0, 0)),
            pl.BlockSpec((1, D), lambda i: (0, 0)),
            pl.BlockSpec((D, NE), lambda i: (0, 0)),
            pl.BlockSpec((1, NE), lambda i: (0, 0)),
        ],
        out_specs=[
            pl.BlockSpec((bm, D), lambda i: (i, 0)),
            pl.BlockSpec((bm, 128), lambda i: (i, 0)),
            pl.BlockSpec((bm, 128), lambda i: (i, 0)),
        ],
        out_shape=[
            jax.ShapeDtypeStruct((N, D), F32),
            jax.ShapeDtypeStruct((N, 128), jnp.int32),
            jax.ShapeDtypeStruct((N, 128), F32),
        ],
    )(z, a1, c1, gate_W, gate_b.reshape(1, NE))


def _tc_moe_mlp(blk_e, x_disp, w1, b1, w2, b2, g_bc, bmc):
    P, D = x_disp.shape
    NE, _, H = w1.shape
    grid = P // bmc

    def body(be_ref, x_ref, w1_ref, b1_ref, w2_ref, b2_ref, g_ref, y_ref):
        x = x_ref[...]
        h = jnp.dot(x, w1_ref[0], preferred_element_type=F32) + b1_ref[0]
        h = 0.5 * h * (1.0 + lax.erf(h * (2.0 ** -0.5)))
        y = jnp.dot(h, w2_ref[0], preferred_element_type=F32) + b2_ref[0]
        y_ref[...] = y * g_ref[:, 0:1]

    grid_spec = pltpu.PrefetchScalarGridSpec(
        num_scalar_prefetch=1,
        grid=(grid,),
        in_specs=[
            pl.BlockSpec((bmc, D), lambda i, be: (i, 0)),
            pl.BlockSpec((1, D, H), lambda i, be: (be[i], 0, 0)),
            pl.BlockSpec((1, 1, H), lambda i, be: (be[i], 0, 0)),
            pl.BlockSpec((1, H, D), lambda i, be: (be[i], 0, 0)),
            pl.BlockSpec((1, 1, D), lambda i, be: (be[i], 0, 0)),
            pl.BlockSpec((bmc, 128), lambda i, be: (i, 0)),
        ],
        out_specs=pl.BlockSpec((bmc, D), lambda i, be: (i, 0)),
    )
    return pl.pallas_call(
        body,
        grid_spec=grid_spec,
        out_shape=jax.ShapeDtypeStruct((P, D), F32),
    )(blk_e, x_disp, w1, b1.reshape(NE, 1, H), w2, b2.reshape(NE, 1, D), g_bc)


def _tc_onehot_agg(blk_nb, first, x_e, dloc3, bme, nb_out):
    P, D = x_e.shape
    NBLK = P // bme

    def body(nb_ref, fr_ref, x_ref, dl_ref, o_ref):
        ids = dl_ref[0].reshape(bme, 1)
        io = lax.broadcasted_iota(jnp.int32, (bme, 256), 1)
        L = (ids == io).astype(F32)
        acc = lax.dot_general(L, x_ref[...], (((0,), (0,)), ((), ())),
                              preferred_element_type=F32,
                              precision=lax.Precision.HIGHEST)
        i = pl.program_id(0)

        @pl.when(fr_ref[i] == 1)
        def _():
            o_ref[...] = acc

        @pl.when(fr_ref[i] == 0)
        def _():
            o_ref[...] = o_ref[...] + acc

    grid_spec = pltpu.PrefetchScalarGridSpec(
        num_scalar_prefetch=2,
        grid=(NBLK,),
        in_specs=[
            pl.BlockSpec((bme, D), lambda i, nb, fr: (i, 0)),
            pl.BlockSpec((1, 1, bme), lambda i, nb, fr: (i, 0, 0)),
        ],
        out_specs=pl.BlockSpec((256, D), lambda i, nb, fr: (nb[i], 0)),
    )
    return pl.pallas_call(
        body,
        grid_spec=grid_spec,
        out_shape=jax.ShapeDtypeStruct((nb_out * 256, D), F32),
    )(blk_nb, first, x_e, dloc3)


def _tc_add_sum(new, v, bm):
    N, D = new.shape
    grid = N // bm

    def body(a_ref, b_ref, c_ref, z_ref, st_ref):
        z = a_ref[...] + b_ref[...] + c_ref[...]
        z_ref[...] = z
        part = jnp.concatenate([jnp.sum(z, axis=0, keepdims=True),
                                jnp.zeros((7, D), F32)], axis=0)

        @pl.when(pl.program_id(0) == 0)
        def _():
            st_ref[...] = part

        @pl.when(pl.program_id(0) > 0)
        def _():
            st_ref[...] = st_ref[...] + part

    return pl.pallas_call(
        body,
        grid=(grid,),
        in_specs=[
            pl.BlockSpec((bm, D), lambda i: (i, 0)),
            pl.BlockSpec((bm, D), lambda i: (i, 0)),
            pl.BlockSpec((bm, D), lambda i, g=grid: (g + i, 0)),
        ],
        out_specs=[
            pl.BlockSpec((bm, D), lambda i: (i, 0)),
            pl.BlockSpec((8, D), lambda i: (0, 0)),
        ],
        out_shape=[
            jax.ShapeDtypeStruct((N, D), F32),
            jax.ShapeDtypeStruct((8, D), F32),
        ],
    )(new, v, v)


def _tc_affine(z, a, c, bm):
    N, D = z.shape
    grid = N // bm

    def body(z_ref, a_ref, c_ref, o_ref):
        o_ref[...] = z_ref[...] * a_ref[...] + c_ref[...]

    return pl.pallas_call(
        body,
        grid=(grid,),
        in_specs=[
            pl.BlockSpec((bm, D), lambda i: (i, 0)),
            pl.BlockSpec((1, D), lambda i: (0, 0)),
            pl.BlockSpec((1, D), lambda i: (0, 0)),
        ],
        out_specs=pl.BlockSpec((bm, D), lambda i: (i, 0)),
        out_shape=jax.ShapeDtypeStruct((N, D), F32),
    )(z, a, c)


def _bn_coeffs(mu, varsum, g, b, n):
    var = varsum[0] / n
    inv = lax.rsqrt(var + EPS)
    a = (g * inv)[None, :]
    c = (b - mu[0] * g * inv)[None, :]
    return a, c


def kernel(feats, edge_index, gc_W, gc_b, res_W, res_b, bn1_g, bn1_b,
           gate_W, gate_b, w1, b1, w2, b2, bn2_g, bn2_b):
    N, D = feats.shape
    NE = gate_W.shape[1]
    K = 2
    BM = 400
    BMC = 128
    P = 20000 + NE * BMC
    P = ((P + 32 * BMC - 1) // (32 * BMC)) * (32 * BMC)

    src = edge_index[0]
    dst = edge_index[1]
    E = src.shape[0]

    BME = 512
    NBQ = -(-N // 256)
    PE = E + NBQ * BME
    PE = -(-PE // 3072) * 3072
    nbkey = dst // 256
    order_e = jnp.argsort(nbkey)
    key_s = nbkey[order_e]
    src_s = src[order_e]
    dloc_s = dst[order_e] - key_s * 256
    cnts = jnp.zeros((NBQ,), jnp.int32).at[nbkey].add(1)
    starts_e = jnp.concatenate([jnp.zeros((1,), jnp.int32),
                                jnp.cumsum(cnts)[:-1]])
    pce = jnp.maximum(-(-cnts // BME), 1) * BME
    pstarts_e = jnp.concatenate([jnp.zeros((1,), jnp.int32),
                                 jnp.cumsum(pce)[:-1]])
    dest_e = pstarts_e[key_s] + jnp.arange(E, dtype=jnp.int32) - starts_e[key_s]
    src_pad = jnp.zeros((PE,), jnp.int32).at[dest_e].set(src_s)
    dloc_pad = jnp.full((PE,), 256, jnp.int32).at[dest_e].set(dloc_s)
    bidx_e = jnp.arange(PE // BME, dtype=jnp.int32) * BME
    blk_nb = jnp.clip(
        jnp.searchsorted(pstarts_e, bidx_e, side="right").astype(jnp.int32) - 1,
        0, NBQ - 1)
    first_e = (bidx_e == pstarts_e[blk_nb]).astype(jnp.int32)

    sup, res = _tc_support_res(feats, gc_W, res_W, res_b, BM)
    x_e = _gather_sc(sup, src_pad)
    agg = _tc_onehot_agg(blk_nb, first_e, x_e,
                         dloc_pad.reshape(PE // BME, 1, BME), BME, NBQ)[:N]

    z1, s1 = _tc_zsum(agg, res, gc_b, BM)
    mu1 = (s1[0] / N)[None, :]
    vs1 = _tc_varsum(z1, mu1, BM)
    a1, c1 = _bn_coeffs(mu1, vs1, bn1_g, bn1_b, N)
    new, ei_out, gv_out = _tc_bn_gate(z1, a1, c1, gate_W, gate_b, BM)

    ei = ei_out[:, :K].reshape(-1)
    gv = gv_out[:, :K].reshape(-1)
    tok = jnp.arange(N * K, dtype=jnp.int32) // K
    order = jnp.argsort(ei)
    es = ei[order]
    toks = tok[order]
    gs = gv[order]
    counts = jnp.zeros((NE,), jnp.int32).at[ei].add(1)
    starts = jnp.concatenate([jnp.zeros((1,), jnp.int32),
                              jnp.cumsum(counts)[:-1]])
    pc = ((counts + BMC - 1) // BMC) * BMC
    pstarts = jnp.concatenate([jnp.zeros((1,), jnp.int32),
                               jnp.cumsum(pc)[:-1]])
    pos = jnp.arange(N * K, dtype=jnp.int32) - starts[es]
    dest = pstarts[es] + pos
    tok_disp = jnp.zeros((P,), jnp.int32).at[dest].set(toks)
    g_disp = jnp.zeros((P,), F32).at[dest].set(gs)
    g_bc = jnp.broadcast_to(g_disp[:, None], (P, 128))
    blk_e = jnp.clip(
        jnp.searchsorted(pstarts, jnp.arange(P // BMC, dtype=jnp.int32) * BMC,
                         side="right").astype(jnp.int32) - 1, 0, NE - 1)

    x_disp = _gather_sc(new, tok_disp)
    y_disp = _tc_moe_mlp(blk_e, x_disp, w1, b1, w2, b2, g_bc, BMC)
    pos_t = jnp.zeros((N * K,), jnp.int32).at[order].set(dest)
    PV = -(-(N * K) // 4096) * 4096
    pos_all = jnp.concatenate(
        [pos_t[0::2], pos_t[1::2],
         jnp.zeros((PV - N * K,), jnp.int32)])
    v = _gather_sc(y_disp, pos_all)

    z2, s2 = _tc_add_sum(new, v, BM)
    mu2 = (s2[0] / N)[None, :]
    vs2 = _tc_varsum(z2, mu2, BM)
    a2, c2 = _bn_coeffs(mu2, vs2, bn2_g, bn2_b, N)
    return _tc_affine(z2, a2, c2, BM)
